# Initial kernel scaffold; baseline (speedup 1.0000x reference)
import functools

import jax
import jax.numpy as jnp
from jax import lax
from jax.experimental import pallas as pl
from jax.experimental.pallas import tpu as pltpu
from jax.experimental.pallas import tpu_sc as plsc

_VOCAB = 32768
_EMBED = 512
_B = 1024 * 128

_info = plsc.get_sparse_core_info()
_NC = _info.num_cores
_NS = _info.num_subcores
_NW = _NC * _NS
_BPW = _B // _NW
_CHUNK = 128
_NCHUNK = _BPW // _CHUNK

_mesh = plsc.VectorSubcoreMesh(core_axis_name="c", subcore_axis_name="s")


@functools.partial(
    pl.kernel,
    mesh=_mesh,
    out_type=jax.ShapeDtypeStruct((_B, _EMBED), jnp.float32),
    scratch_types=[
        pltpu.VMEM((_BPW,), jnp.int32),
        pltpu.VMEM((_CHUNK, _EMBED), jnp.float32),
        pltpu.SemaphoreType.DMA,
    ],
)
def _emb_lookup(x_hbm, table_hbm, out_hbm, idx_v, rows_v, sem):
    wid = lax.axis_index("s") * _NC + lax.axis_index("c")
    base = wid * _BPW
    pltpu.sync_copy(x_hbm.at[pl.ds(base, _BPW)], idx_v)

    def body(c, _):
        idxs = idx_v.at[pl.ds(c * _CHUNK, _CHUNK)]
        pltpu.async_copy(table_hbm.at[idxs], rows_v, sem).wait()
        pltpu.sync_copy(rows_v, out_hbm.at[pl.ds(base + c * _CHUNK, _CHUNK)])

    pl.loop(0, _NCHUNK)(body)


def kernel(x, table):
    flat = x.reshape(_B)
    out = _emb_lookup(flat, table)
    return out.reshape(x.shape[0], x.shape[1], _EMBED)


# speedup vs baseline: 1.9707x; 1.9707x over previous
import functools

import jax
import jax.numpy as jnp
from jax import lax
from jax.experimental import pallas as pl
from jax.experimental.pallas import tpu as pltpu
from jax.experimental.pallas import tpu_sc as plsc

_VOCAB = 32768
_EMBED = 512
_B = 1024 * 128

_info = plsc.get_sparse_core_info()
_NC = _info.num_cores
_NS = _info.num_subcores
_NW = _NC * _NS
_BPW = _B // _NW
_CHUNK = 128
_NCHUNK = _BPW // _CHUNK

_mesh = plsc.VectorSubcoreMesh(core_axis_name="c", subcore_axis_name="s")


@functools.partial(
    pl.kernel,
    mesh=_mesh,
    out_type=jax.ShapeDtypeStruct((_B, _EMBED), jnp.float32),
    scratch_types=[
        pltpu.VMEM((_BPW,), jnp.int32),
        pltpu.VMEM((_CHUNK, _EMBED), jnp.float32),
        pltpu.SemaphoreType.DMA,
    ],
)
def _emb_lookup(x_hbm, table_hbm, out_hbm, idx_v, rows_v, sem):
    wid = lax.axis_index("s") * _NC + lax.axis_index("c")
    base = wid * _BPW
    pltpu.sync_copy(x_hbm.at[pl.ds(base, _BPW)], idx_v)

    def body(c):
        idxs = idx_v.at[pl.ds(c * _CHUNK, _CHUNK)]
        pltpu.async_copy(table_hbm.at[idxs], rows_v, sem).wait()
        pltpu.sync_copy(rows_v, out_hbm.at[pl.ds(base + c * _CHUNK, _CHUNK)])

    pl.loop(0, _NCHUNK)(body)


def kernel(x, table):
    flat = x.reshape(_B)
    out = _emb_lookup(flat, table)
    return out.reshape(x.shape[0], x.shape[1], _EMBED)


# speedup vs baseline: 2.1534x; 1.0927x over previous
import functools

import jax
import jax.numpy as jnp
from jax import lax
from jax.experimental import pallas as pl
from jax.experimental.pallas import tpu as pltpu
from jax.experimental.pallas import tpu_sc as plsc

_VOCAB = 32768
_EMBED = 512
_B = 1024 * 128

_info = plsc.get_sparse_core_info()
_NC = _info.num_cores
_NS = _info.num_subcores
_NW = _NC * _NS
_BPW = _B // _NW
_CHUNK = 64
_NCHUNK = _BPW // _CHUNK

_mesh = plsc.VectorSubcoreMesh(core_axis_name="c", subcore_axis_name="s")


@functools.partial(
    pl.kernel,
    mesh=_mesh,
    out_type=jax.ShapeDtypeStruct((_B, _EMBED), jnp.float32),
    scratch_types=[
        pltpu.VMEM((_BPW,), jnp.int32),
        pltpu.VMEM((2, _CHUNK, _EMBED), jnp.float32),
        pltpu.SemaphoreType.DMA,
        pltpu.SemaphoreType.DMA,
    ],
)
def _emb_lookup(x_hbm, table_hbm, out_hbm, idx_v, rows_v, gsem, ssem):
    wid = lax.axis_index("s") * _NC + lax.axis_index("c")
    base = wid * _BPW
    pltpu.sync_copy(x_hbm.at[pl.ds(base, _BPW)], idx_v)

    def gather(c, b):
        idxs = idx_v.at[pl.ds(c * _CHUNK, _CHUNK)]
        pltpu.async_copy(table_hbm.at[idxs], rows_v.at[b], gsem)

    def wait_gather(b):
        pltpu.make_async_copy(
            table_hbm.at[idx_v.at[pl.ds(0, _CHUNK)]], rows_v.at[b], gsem
        ).wait()

    def scatter(c, b):
        pltpu.async_copy(
            rows_v.at[b], out_hbm.at[pl.ds(base + c * _CHUNK, _CHUNK)], ssem
        )

    def wait_scatter(b):
        pltpu.make_async_copy(
            rows_v.at[b], out_hbm.at[pl.ds(base, _CHUNK)], ssem
        ).wait()

    gather(0, 0)
    gather(1, 1)
    wait_gather(0)
    scatter(0, 0)

    def body(g):
        for d in range(2):
            c = g + d
            b = (1 + d) % 2
            wait_scatter(1 - b)
            gather(c + 1, 1 - b)
            wait_gather(b)
            scatter(c, b)

    pl.loop(1, _NCHUNK - 1, step=2)(body)

    b_last = (_NCHUNK - 1) % 2
    wait_scatter(1 - b_last)
    wait_gather(b_last)
    scatter(_NCHUNK - 1, b_last)
    wait_scatter(b_last)


def kernel(x, table):
    flat = x.reshape(_B)
    out = _emb_lookup(flat, table)
    return out.reshape(x.shape[0], x.shape[1], _EMBED)


# speedup vs baseline: 2.1643x; 1.0051x over previous
import functools

import jax
import jax.numpy as jnp
from jax import lax
from jax.experimental import pallas as pl
from jax.experimental.pallas import tpu as pltpu
from jax.experimental.pallas import tpu_sc as plsc

_VOCAB = 32768
_EMBED = 512
_B = 1024 * 128

_info = plsc.get_sparse_core_info()
_NC = _info.num_cores
_NS = _info.num_subcores
_NW = _NC * _NS
_BPW = _B // _NW
_CHUNK = 64
_NCHUNK = _BPW // _CHUNK

_mesh = plsc.VectorSubcoreMesh(core_axis_name="c", subcore_axis_name="s")


@functools.partial(
    pl.kernel,
    mesh=_mesh,
    out_type=jax.ShapeDtypeStruct((_B, _EMBED), jnp.float32),
    scratch_types=[
        pltpu.VMEM((_BPW,), jnp.int32),
        pltpu.VMEM((3, _CHUNK, _EMBED), jnp.float32),
        pltpu.SemaphoreType.DMA,
        pltpu.SemaphoreType.DMA,
    ],
)
def _emb_lookup(x_hbm, table_hbm, out_hbm, idx_v, rows_v, gsem, ssem):
    wid = lax.axis_index("s") * _NC + lax.axis_index("c")
    base = wid * _BPW
    pltpu.sync_copy(x_hbm.at[pl.ds(base, _BPW)], idx_v)

    def gather(c, b):
        idxs = idx_v.at[pl.ds(c * _CHUNK, _CHUNK)]
        pltpu.async_copy(table_hbm.at[idxs], rows_v.at[b], gsem)

    def wait_gather(b):
        pltpu.make_async_copy(
            table_hbm.at[idx_v.at[pl.ds(0, _CHUNK)]], rows_v.at[b], gsem
        ).wait()

    def scatter(c, b):
        pltpu.async_copy(
            rows_v.at[b], out_hbm.at[pl.ds(base + c * _CHUNK, _CHUNK)], ssem
        )

    def wait_scatter(b):
        pltpu.make_async_copy(
            rows_v.at[b], out_hbm.at[pl.ds(base, _CHUNK)], ssem
        ).wait()

    gather(0, 0)
    gather(1, 1)

    wait_gather(0)
    scatter(0, 0)
    gather(2, 2)
    wait_gather(1)
    scatter(1, 1)
    wait_scatter(0)
    gather(3, 0)

    def body(g):
        for d in range(3):
            c = g + d
            b = (2 + d) % 3
            wait_gather(b)
            scatter(c, b)
            wait_scatter((b + 2) % 3)
            gather(c + 2, (b + 2) % 3)

    pl.loop(2, _NCHUNK - 2, step=3)(body)

    b = (_NCHUNK - 2) % 3
    wait_gather(b)
    scatter(_NCHUNK - 2, b)
    wait_scatter((b + 2) % 3)
    b = (_NCHUNK - 1) % 3
    wait_gather(b)
    scatter(_NCHUNK - 1, b)
    wait_scatter((b + 2) % 3)
    wait_scatter(b)


def kernel(x, table):
    flat = x.reshape(_B)
    out = _emb_lookup(flat, table)
    return out.reshape(x.shape[0], x.shape[1], _EMBED)
